# baseline (speedup 1.0000x reference)
import functools

import jax
import jax.numpy as jnp
from jax.experimental import pallas as pl
from jax.experimental.pallas import tpu as pltpu

E_BLOCK = 8000
N_BLOCK = 10000


def _edge_mlp_body(xj_ref, ea_ref, w1ax_ref, w1ae_ref, b1a_ref, w1b_ref,
                   b1b_ref, out_ref):
    h = jnp.dot(xj_ref[...], w1ax_ref[...], preferred_element_type=jnp.float32)
    h = h + jnp.dot(ea_ref[...], w1ae_ref[...],
                    preferred_element_type=jnp.float32)
    h = jax.nn.relu(h + b1a_ref[...])
    msg = jnp.dot(h, w1b_ref[...], preferred_element_type=jnp.float32)
    out_ref[...] = jax.nn.relu(msg + b1b_ref[...])


def _node_mlp_body(x_ref, aggr_ref, w2ax_ref, w2aa_ref, b2a_ref, w2b_ref,
                   b2b_ref, out_ref):
    x = x_ref[...]
    h2 = jnp.dot(x, w2ax_ref[...], preferred_element_type=jnp.float32)
    h2 = h2 + jnp.dot(aggr_ref[...], w2aa_ref[...],
                      preferred_element_type=jnp.float32)
    h2 = jax.nn.relu(h2 + b2a_ref[...])
    comb_all = jnp.dot(h2, w2b_ref[...], preferred_element_type=jnp.float32)
    comb_all = comb_all + b2b_ref[...]
    links = comb_all[:, 0:1]
    comb = comb_all[:, 1:5]
    nor = jnp.sqrt(jnp.sum(comb * comb, axis=1, keepdims=True))
    comb = comb / jnp.maximum(jnp.ones_like(nor), nor)
    out_ref[...] = jnp.concatenate([links, comb, x[:, :3]], axis=1)


def _edge_mlp(xj, ea, w1ax, w1ae, b1a, w1b, b1b):
    e = xj.shape[0]
    grid = (e // E_BLOCK,)
    full = lambda i: (0, 0)
    return pl.pallas_call(
        _edge_mlp_body,
        grid=grid,
        in_specs=[
            pl.BlockSpec((E_BLOCK, 8), lambda i: (i, 0)),
            pl.BlockSpec((E_BLOCK, 5), lambda i: (i, 0)),
            pl.BlockSpec((8, 64), full),
            pl.BlockSpec((5, 64), full),
            pl.BlockSpec((1, 64), full),
            pl.BlockSpec((64, 64), full),
            pl.BlockSpec((1, 64), full),
        ],
        out_specs=pl.BlockSpec((E_BLOCK, 64), lambda i: (i, 0)),
        out_shape=jax.ShapeDtypeStruct((e, 64), jnp.float32),
    )(xj, ea, w1ax, w1ae, b1a, w1b, b1b)


def _node_mlp(x, aggr, w2ax, w2aa, b2a, w2b, b2b):
    n = x.shape[0]
    grid = (n // N_BLOCK,)
    full = lambda i: (0, 0)
    return pl.pallas_call(
        _node_mlp_body,
        grid=grid,
        in_specs=[
            pl.BlockSpec((N_BLOCK, 8), lambda i: (i, 0)),
            pl.BlockSpec((N_BLOCK, 64), lambda i: (i, 0)),
            pl.BlockSpec((8, 32), full),
            pl.BlockSpec((64, 32), full),
            pl.BlockSpec((1, 32), full),
            pl.BlockSpec((32, 5), full),
            pl.BlockSpec((1, 5), full),
        ],
        out_specs=pl.BlockSpec((N_BLOCK, 8), lambda i: (i, 0)),
        out_shape=jax.ShapeDtypeStruct((n, 8), jnp.float32),
    )(x, aggr, w2ax, w2aa, b2a, w2b, b2b)


def kernel(x, edge_index, edge_attr, W1a, b1a, W1b, b1b, W2a, b2a, W2b, b2b):
    src = edge_index[0]
    dst = edge_index[1]
    n = x.shape[0]

    w1ax, w1ae = W1a[:8], W1a[8:]
    w2ax, w2aa = W2a[:8], W2a[8:]
    b1a2 = b1a.reshape(1, 64)
    b1b2 = b1b.reshape(1, 64)
    b2a2 = b2a.reshape(1, 32)
    b2b2 = b2b.reshape(1, 5)

    def layer(x):
        xj = jnp.take(x, src, axis=0)
        msg = _edge_mlp(xj, edge_attr, w1ax, w1ae, b1a2, W1b, b1b2)
        aggr = jax.ops.segment_max(msg, dst, num_segments=n)
        aggr = jnp.where(jnp.isfinite(aggr), aggr, 0.0)
        return _node_mlp(x, aggr, w2ax, w2aa, b2a2, W2b, b2b2)

    for _ in range(8):
        x = layer(x)
    return x
